# Initial kernel scaffold; baseline (speedup 1.0000x reference)
import jax
import jax.numpy as jnp
from jax.experimental import pallas as pl


def kernel(x, edge_index, W1, b1, beta2, W2, b2):
    raise NotImplementedError("write your pallas kernel here")


# speedup vs baseline: 1.9363x; 1.9363x over previous
import jax
import jax.numpy as jnp
from jax.experimental import pallas as pl


def _lin_relu_body(x_ref, wt_ref, b_ref, o_ref):
    acc = jnp.dot(x_ref[...], wt_ref[...],
                  preferred_element_type=jnp.float32,
                  precision=jax.lax.Precision.HIGHEST)
    o_ref[...] = jnp.maximum(acc + b_ref[...], 0.0)


def _lin_relu(x, Wt, b):
    n, d_in = x.shape
    d_out = Wt.shape[1]
    blk = 1000
    return pl.pallas_call(
        _lin_relu_body,
        grid=(n // blk,),
        in_specs=[pl.BlockSpec((blk, d_in), lambda i: (i, 0)),
                  pl.BlockSpec((d_in, d_out), lambda i: (0, 0)),
                  pl.BlockSpec((1, d_out), lambda i: (0, 0))],
        out_specs=pl.BlockSpec((blk, d_out), lambda i: (i, 0)),
        out_shape=jax.ShapeDtypeStruct((n, d_out), jnp.float32),
    )(x, Wt, b[None, :])


def _agnn_jax(x, src, dst, beta, num_nodes):
    nrm = jnp.linalg.norm(x, axis=-1, keepdims=True)
    x_norm = x / jnp.maximum(nrm, 1e-12)
    alpha = beta * jnp.sum(x_norm[dst] * x_norm[src], axis=-1)
    ex = jnp.exp(alpha)
    denom = jax.ops.segment_sum(ex, dst, num_segments=num_nodes)
    num = jax.ops.segment_sum(x[src] * ex[:, None], dst, num_segments=num_nodes)
    return num / (denom[:, None] + 1e-16)


def _lin_logsoftmax_body(x_ref, wt_ref, b_ref, o_ref):
    acc = jnp.dot(x_ref[...], wt_ref[...],
                  preferred_element_type=jnp.float32,
                  precision=jax.lax.Precision.HIGHEST) + b_ref[...]
    m = jnp.max(acc, axis=-1, keepdims=True)
    lse = jnp.log(jnp.sum(jnp.exp(acc - m), axis=-1, keepdims=True)) + m
    o_ref[...] = acc - lse


def _lin_logsoftmax(x, Wt, b):
    n, d_in = x.shape
    d_out = Wt.shape[1]
    blk = 1000
    return pl.pallas_call(
        _lin_logsoftmax_body,
        grid=(n // blk,),
        in_specs=[pl.BlockSpec((blk, d_in), lambda i: (i, 0)),
                  pl.BlockSpec((d_in, d_out), lambda i: (0, 0)),
                  pl.BlockSpec((1, d_out), lambda i: (0, 0))],
        out_specs=pl.BlockSpec((blk, d_out), lambda i: (i, 0)),
        out_shape=jax.ShapeDtypeStruct((n, d_out), jnp.float32),
    )(x, Wt, b[None, :])


def kernel(x, edge_index, W1, b1, beta2, W2, b2):
    num_nodes = x.shape[0]
    loop = jnp.arange(num_nodes)
    src = jnp.concatenate([edge_index[0], loop])
    dst = jnp.concatenate([edge_index[1], loop])
    h = _lin_relu(x, W1.T, b1)
    h1 = _agnn_jax(h, src, dst, 1.0, num_nodes)
    h2 = _agnn_jax(h1, src, dst, beta2[0], num_nodes)
    logp = _lin_logsoftmax(h2, W2.T, b2)
    return (h1, h2, logp)


# speedup vs baseline: 6.7536x; 3.4880x over previous
import functools

import jax
import jax.numpy as jnp
from jax import lax
from jax.experimental import pallas as pl
from jax.experimental.pallas import tpu as pltpu
from jax.experimental.pallas import tpu_sc as plsc

N_NODES = 10000
D = 128
AUG = 144
N_TILES = 32
PER_TILE = 10368
E_PAD = N_TILES * PER_TILE
CHUNK = 128
N_CHUNK = PER_TILE // CHUNK
N_PAD = 10240
ROWS_PER_TILE = N_PAD // 16


def _front_body(x_ref, wt_ref, b_ref, aug_ref):
    h = jnp.dot(x_ref[...], wt_ref[...],
                preferred_element_type=jnp.float32,
                precision=jax.lax.Precision.HIGHEST)
    h = jnp.maximum(h + b_ref[...], 0.0)
    nrm = jnp.sqrt(jnp.sum(h * h, axis=-1, keepdims=True))
    invn = 1.0 / jnp.maximum(nrm, 1e-12)
    aug_ref[:, :D] = h
    cols = lax.broadcasted_iota(jnp.int32, (h.shape[0], AUG - D), 1)
    aug_ref[:, D:] = jnp.where(cols <= 1, invn, 0.0)


def _front(x, W1t, b1):
    n = x.shape[0]
    blk = 1000
    return pl.pallas_call(
        _front_body,
        grid=(n // blk,),
        in_specs=[pl.BlockSpec((blk, D), lambda i: (i, 0)),
                  pl.BlockSpec((D, D), lambda i: (0, 0)),
                  pl.BlockSpec((1, D), lambda i: (0, 0))],
        out_specs=pl.BlockSpec((blk, AUG), lambda i: (i, 0)),
        out_shape=jax.ShapeDtypeStruct((n, AUG), jnp.float32),
    )(x, W1t, b1[None, :])


def _mid_body(acc_ref, beta_ref, z_ref, aug_ref):
    s = acc_ref[0] + acc_ref[1]
    denom = s[:, D:D + 1]
    h = s[:, :D] / (denom + 1e-16)
    z_ref[...] = h
    nrm = jnp.sqrt(jnp.sum(h * h, axis=-1, keepdims=True))
    invn = 1.0 / jnp.maximum(nrm, 1e-12)
    aug_ref[:, :D] = h
    cols = lax.broadcasted_iota(jnp.int32, (h.shape[0], AUG - D), 1)
    invn_b = jnp.where(cols == 1, invn * beta_ref[0, 0], invn)
    aug_ref[:, D:] = jnp.where(cols <= 1, invn_b, 0.0)


def _mid(acc, beta2):
    n = N_NODES
    blk = 1000
    return pl.pallas_call(
        _mid_body,
        grid=(n // blk,),
        in_specs=[pl.BlockSpec((2, blk, AUG), lambda i: (0, i, 0)),
                  pl.BlockSpec((1, 1), lambda i: (0, 0))],
        out_specs=[pl.BlockSpec((blk, D), lambda i: (i, 0)),
                   pl.BlockSpec((blk, AUG), lambda i: (i, 0))],
        out_shape=[jax.ShapeDtypeStruct((n, D), jnp.float32),
                   jax.ShapeDtypeStruct((n, AUG), jnp.float32)],
    )(acc, beta2)


def _final_body(acc_ref, wt_ref, b_ref, z_ref, lp_ref):
    s = acc_ref[0] + acc_ref[1]
    denom = s[:, D:D + 1]
    h = s[:, :D] / (denom + 1e-16)
    z_ref[...] = h
    logits = jnp.dot(h, wt_ref[...],
                     preferred_element_type=jnp.float32,
                     precision=jax.lax.Precision.HIGHEST) + b_ref[...]
    m = jnp.max(logits, axis=-1, keepdims=True)
    lse = jnp.log(jnp.sum(jnp.exp(logits - m), axis=-1, keepdims=True)) + m
    lp_ref[...] = logits - lse


def _final(acc, W2t, b2):
    n = N_NODES
    d_out = W2t.shape[1]
    blk = 1000
    return pl.pallas_call(
        _final_body,
        grid=(n // blk,),
        in_specs=[pl.BlockSpec((2, blk, AUG), lambda i: (0, i, 0)),
                  pl.BlockSpec((D, d_out), lambda i: (0, 0)),
                  pl.BlockSpec((1, d_out), lambda i: (0, 0))],
        out_specs=[pl.BlockSpec((blk, D), lambda i: (i, 0)),
                   pl.BlockSpec((blk, d_out), lambda i: (i, 0))],
        out_shape=[jax.ShapeDtypeStruct((n, D), jnp.float32),
                   jax.ShapeDtypeStruct((n, d_out), jnp.float32)],
    )(acc, W2t, b2[None, :])


def _sc_body(aug_hbm, src_hbm, dst_hbm, val_hbm, zeros_hbm, acc_hbm,
             acc_sh, sidx, didx, vbuf, exbuf, sbuf, dbuf,
             sem0, sem1):
    cid = lax.axis_index("c")
    sid = lax.axis_index("s")
    wid = cid * 16 + sid
    base = wid * PER_TILE
    row0 = sid * ROWS_PER_TILE

    pltpu.sync_copy(zeros_hbm, acc_sh.at[pl.ds(row0, ROWS_PER_TILE)])
    plsc.subcore_barrier()

    lanes = lax.iota(jnp.int32, 16)
    zeros16 = jnp.zeros((16,), jnp.float32)
    zeros16i = jnp.zeros((16,), jnp.int32)

    def chunk_body(ci, _):
        cb = base + ci * CHUNK
        pltpu.sync_copy(src_hbm.at[pl.ds(cb, CHUNK)], sidx)
        pltpu.sync_copy(dst_hbm.at[pl.ds(cb, CHUNK)], didx)
        pltpu.sync_copy(val_hbm.at[pl.ds(cb, CHUNK)], vbuf)
        cp0 = pltpu.async_copy(aug_hbm.at[sidx], sbuf, sem0)
        cp1 = pltpu.async_copy(aug_hbm.at[didx], dbuf, sem1)
        cp0.wait()
        cp1.wait()

        for g in range(CHUNK // 16):
            riv = lanes + (g * 16)

            def dim_body(d, carry):
                acc, civ = carry
                s = plsc.load_gather(sbuf, [riv, civ])
                t = plsc.load_gather(dbuf, [riv, civ])
                return acc + s * t, civ + 1

            dotv, _ = lax.fori_loop(0, D, dim_body, (zeros16, zeros16i),
                                    unroll=8)
            invn_s = plsc.load_gather(sbuf, [riv, zeros16i + D])
            invn_db = plsc.load_gather(dbuf, [riv, zeros16i + (D + 1)])
            alpha = dotv * invn_s * invn_db
            ex = jnp.exp(alpha) * vbuf[pl.ds(g * 16, 16)]
            exbuf[pl.ds(g * 16, 16)] = ex

        def scale_body(j, _):
            exs = plsc.load_gather(exbuf, [zeros16i + j])
            for k in range(D // 16):
                sl = pl.ds(k * 16, 16)
                sbuf[j, sl] = sbuf[j, sl] * exs
            sbuf[j, pl.ds(D, 16)] = jnp.where(lanes == 0, exs, 0.0)
            return 0

        lax.fori_loop(0, CHUNK, scale_body, 0, unroll=4)

        pltpu.sync_copy(sbuf, acc_sh.at[didx], add=True)
        return 0

    lax.fori_loop(0, N_CHUNK, chunk_body, 0)

    plsc.subcore_barrier()
    pltpu.sync_copy(acc_sh.at[pl.ds(row0, ROWS_PER_TILE)],
                    acc_hbm.at[cid, pl.ds(row0, ROWS_PER_TILE)])


@functools.partial(jax.jit, static_argnames=())
def _sc_layer(aug, src, dst, valid, zeros_slab):
    mesh = plsc.VectorSubcoreMesh(core_axis_name="c", subcore_axis_name="s")
    kern = pl.kernel(
        _sc_body,
        out_type=jax.ShapeDtypeStruct((2, N_PAD, AUG), jnp.float32),
        mesh=mesh,
        compiler_params=pltpu.CompilerParams(use_tc_tiling_on_sc=False,
                                             needs_layout_passes=False),
        scratch_types=[
            pltpu.VMEM_SHARED((N_PAD, AUG), jnp.float32),
            pltpu.VMEM((CHUNK,), jnp.int32),
            pltpu.VMEM((CHUNK,), jnp.int32),
            pltpu.VMEM((CHUNK,), jnp.float32),
            pltpu.VMEM((CHUNK,), jnp.float32),
            pltpu.VMEM((CHUNK, AUG), jnp.float32),
            pltpu.VMEM((CHUNK, AUG), jnp.float32),
            pltpu.SemaphoreType.DMA,
            pltpu.SemaphoreType.DMA,
        ],
    )
    return kern(aug, src, dst, valid, zeros_slab)


def kernel(x, edge_index, W1, b1, beta2, W2, b2):
    num_nodes = x.shape[0]
    n_edges = edge_index.shape[1]
    e_total = n_edges + num_nodes
    loop = jnp.arange(num_nodes, dtype=jnp.int32)
    pad = E_PAD - e_total
    src = jnp.concatenate([edge_index[0].astype(jnp.int32), loop,
                           jnp.zeros((pad,), jnp.int32)])
    dst = jnp.concatenate([edge_index[1].astype(jnp.int32), loop,
                           jnp.zeros((pad,), jnp.int32)])
    valid = jnp.concatenate([jnp.ones((e_total,), jnp.float32),
                             jnp.zeros((pad,), jnp.float32)])
    zeros_slab = jnp.zeros((ROWS_PER_TILE, AUG), jnp.float32)

    aug0 = _front(x, W1.T, b1)
    acc1 = _sc_layer(aug0, src, dst, valid, zeros_slab)
    z1, aug1 = _mid(acc1, beta2.reshape(1, 1))
    acc2 = _sc_layer(aug1, src, dst, valid, zeros_slab)
    z2, logp = _final(acc2, W2.T, b2)
    return (z1, z2, logp)
